# baseline (speedup 1.0000x reference)
import functools

import jax
import jax.numpy as jnp
from jax import lax
from jax.experimental import pallas as pl
from jax.experimental.pallas import tpu as pltpu
from jax.experimental.pallas import tpu_sc as plsc

K = 32
QC = 4


def _split3(M):
    m1 = M.astype(jnp.bfloat16)
    r1 = M - m1.astype(jnp.float32)
    m2 = r1.astype(jnp.bfloat16)
    m3 = (r1 - m2.astype(jnp.float32)).astype(jnp.bfloat16)
    return m1, m2, m3


def _dotf(a, b):
    return jnp.dot(a, b, preferred_element_type=jnp.float32)


def _fps_body(x_ref, y_ref, z_ref, qx_ref, qy_ref, qz_ref):
    X = x_ref[...]
    Y = y_ref[...]
    Z = z_ref[...]
    B, N = X.shape
    S = qx_ref.shape[1]
    iotaN = jax.lax.broadcasted_iota(jnp.int32, (B, N), 1)
    iotaS = jax.lax.broadcasted_iota(jnp.int32, (B, S), 1)

    def step(i, carry):
        dists, far, qx, qy, qz = carry
        onehot = iotaN == far
        cx = jnp.sum(jnp.where(onehot, X, 0.0), axis=1, keepdims=True)
        cy = jnp.sum(jnp.where(onehot, Y, 0.0), axis=1, keepdims=True)
        cz = jnp.sum(jnp.where(onehot, Z, 0.0), axis=1, keepdims=True)
        d = (X - cx) ** 2 + (Y - cy) ** 2 + (Z - cz) ** 2
        dists = jnp.minimum(dists, d)
        sel = iotaS == i
        qx = jnp.where(sel, cx, qx)
        qy = jnp.where(sel, cy, qy)
        qz = jnp.where(sel, cz, qz)
        m = jnp.max(dists, axis=1, keepdims=True)
        far = jnp.min(jnp.where(dists == m, iotaN, N), axis=1, keepdims=True)
        return dists, far, qx, qy, qz

    dists0 = jnp.full((B, N), 1e10, dtype=jnp.float32)
    far0 = jnp.zeros((B, 1), dtype=jnp.int32)
    q0 = jnp.zeros((B, S), dtype=jnp.float32)
    _, _, qx, qy, qz = jax.lax.fori_loop(0, S, step, (dists0, far0, q0, q0, q0))
    qx_ref[...] = qx
    qy_ref[...] = qy
    qz_ref[...] = qz


def _fps_planes(X, Y, Z, npoints):
    B, N = X.shape
    BB = B // 2
    shape = jax.ShapeDtypeStruct((B, npoints), jnp.float32)
    return pl.pallas_call(
        _fps_body,
        grid=(2,),
        in_specs=[pl.BlockSpec((BB, N), lambda i: (i, 0))] * 3,
        out_specs=[pl.BlockSpec((BB, npoints), lambda i: (i, 0))] * 3,
        out_shape=(shape, shape, shape),
        compiler_params=pltpu.CompilerParams(
            dimension_semantics=("parallel",)),
    )(X, Y, Z)


def _sel_body(p_ref, x_ref, y_ref, z_ref, qx_ref, qy_ref, qz_ref,
              w_ref, b_ref, a_ref, gidx_ref, cc_ref):
    P = p_ref[0]
    X = x_ref[0]
    Y = y_ref[0]
    Z = z_ref[0]
    QX = qx_ref[0]
    QY = qy_ref[0]
    QZ = qz_ref[0]
    W = w_ref[...]
    bvec = b_ref[...]
    S = QX.shape[0]
    N = X.shape[1]

    P1, P2, P3 = _split3(P)
    W1, W2, W3 = _split3(W)
    A = (_dotf(P1, W1)
         + (_dotf(P1, W2) + _dotf(P2, W1))
         + (_dotf(P1, W3) + _dotf(P2, W2) + _dotf(P3, W1)))
    a_ref[0] = A

    D0 = (QX - X) ** 2 + (QY - Y) ** 2 + (QZ - Z) ** 2
    CH = 128
    NCH = N // CH
    lane_iota = jax.lax.broadcasted_iota(jnp.int32, (S, CH), 1)
    kiota = jax.lax.broadcasted_iota(jnp.int32, (S, K), 1)

    def step(k, carry):
        m_prev, i_prev, IDX = carry
        mv = jnp.full((S, CH), 1e30, dtype=jnp.float32)
        mi = jnp.full((S, CH), N, dtype=jnp.int32)
        for j in range(NCH):
            Dc = jax.lax.slice_in_dim(D0, j * CH, (j + 1) * CH, axis=1)
            ic = lane_iota + (j * CH)
            a = (Dc > m_prev) | ((Dc == m_prev) & (ic > i_prev))
            dm = jnp.where(a, Dc, 1e30)
            upd = dm < mv
            mv = jnp.where(upd, dm, mv)
            mi = jnp.where(upd, ic, mi)
        m = jnp.min(mv, axis=1, keepdims=True)
        idx = jnp.min(jnp.where(mv == m, mi, N), axis=1, keepdims=True)
        IDX = jnp.where(kiota == k, idx, IDX)
        return m, idx, IDX

    IDX0 = jnp.zeros((S, K), dtype=jnp.int32)
    m0 = jnp.full((S, 1), -1.0, dtype=jnp.float32)
    i0 = jnp.full((S, 1), -1, dtype=jnp.int32)
    _, _, IDX = jax.lax.fori_loop(0, K, step, (m0, i0, IDX0))

    b_id = pl.program_id(0)
    gidx_ref[0] = IDX + b_id * N
    C = QX * W[0:1, :] + QY * W[1:2, :] + QZ * W[2:3, :]
    cc_ref[0] = C - bvec


def _pointconv_select(xp, yp, zp, featP, qx, qy, qz, W, b):
    B, _, N = xp.shape
    S = qx.shape[1]
    Cin = featP.shape[2]
    SB = min(S, 128)
    b2d = b.reshape(1, 64)
    return pl.pallas_call(
        _sel_body,
        grid=(B, S // SB),
        in_specs=[
            pl.BlockSpec((1, N, Cin), lambda i, s: (i, 0, 0)),
            pl.BlockSpec((1, 1, N), lambda i, s: (i, 0, 0)),
            pl.BlockSpec((1, 1, N), lambda i, s: (i, 0, 0)),
            pl.BlockSpec((1, 1, N), lambda i, s: (i, 0, 0)),
            pl.BlockSpec((1, SB, 1), lambda i, s: (i, s, 0)),
            pl.BlockSpec((1, SB, 1), lambda i, s: (i, s, 0)),
            pl.BlockSpec((1, SB, 1), lambda i, s: (i, s, 0)),
            pl.BlockSpec((Cin, 64), lambda i, s: (0, 0)),
            pl.BlockSpec((1, 64), lambda i, s: (0, 0)),
        ],
        out_specs=[
            pl.BlockSpec((1, N, 64), lambda i, s: (i, 0, 0)),
            pl.BlockSpec((1, SB, K), lambda i, s: (i, s, 0)),
            pl.BlockSpec((1, SB, 64), lambda i, s: (i, s, 0)),
        ],
        out_shape=[
            jax.ShapeDtypeStruct((B, N, 64), jnp.float32),
            jax.ShapeDtypeStruct((B, S, K), jnp.int32),
            jax.ShapeDtypeStruct((B, S, 64), jnp.float32),
        ],
        compiler_params=pltpu.CompilerParams(
            dimension_semantics=("parallel", "parallel")),
    )(featP, xp, yp, zp, qx, qy, qz, W, b2d)


def _make_sc_gather_max(S):
    info = plsc.get_sparse_core_info()
    NC = info.num_cores
    G = S // QC
    mesh = plsc.VectorSubcoreMesh(core_axis_name="c", subcore_axis_name="s")

    @functools.partial(
        pl.kernel, mesh=mesh,
        compiler_params=pltpu.CompilerParams(use_tc_tiling_on_sc=False),
        out_type=jax.ShapeDtypeStruct((32, S, 64), jnp.float32),
        scratch_types=[
            pltpu.VMEM((G, QC * K), jnp.int32),
            pltpu.VMEM((QC * K, 64), jnp.float32),
            pltpu.VMEM((S, 64), jnp.float32),
            pltpu.VMEM((S, 64), jnp.float32),
            pltpu.SemaphoreType.DMA,
        ],
    )
    def k(a_hbm, idx_hbm, cc_hbm, out_hbm, idx_v, rows_v, cc_v, out_v, sem):
        w = lax.axis_index("s") * NC + lax.axis_index("c")
        pltpu.sync_copy(idx_hbm.at[w], idx_v)
        pltpu.sync_copy(cc_hbm.at[w], cc_v)

        def chunk(g, carry):
            pltpu.async_copy(a_hbm.at[idx_v.at[g]], rows_v, sem).wait()
            for q in range(QC):
                for c in range(4):
                    ds = pl.ds(c * 16, 16)
                    acc = rows_v[q * K, ds]
                    for kk in range(1, K):
                        acc = jnp.maximum(acc, rows_v[q * K + kk, ds])
                    qq = g * QC + q
                    out_v[qq, ds] = jnp.maximum(acc - cc_v[qq, ds], 0.0)
            return carry

        lax.fori_loop(0, G, chunk, 0)
        pltpu.sync_copy(out_v, out_hbm.at[w])

    return k


def _pool_head_kernel(f3_ref, wf_ref, bf_ref, out_ref):
    pooled = jnp.mean(f3_ref[...], axis=1)
    out_ref[...] = jnp.dot(pooled, wf_ref[...],
                           precision=jax.lax.Precision.HIGHEST) + bf_ref[0]


def _level(xp, yp, zp, featP, qx, qy, qz, W, b):
    B, _, N = xp.shape
    S = qx.shape[1]
    A, gidx, cc = _pointconv_select(xp, yp, zp, featP, qx, qy, qz, W, b)
    idx3 = gidx.reshape(B, S // QC, QC * K)
    F = _make_sc_gather_max(S)(A.reshape(B * N, 64), idx3, cc)
    return F


def kernel(xyz, W1, b1, W2, b2, W3, b3, Wf, bf, faces):
    B, N, _ = xyz.shape
    X, Y, Z = xyz[..., 0], xyz[..., 1], xyz[..., 2]

    qx1, qy1, qz1 = _fps_planes(X, Y, Z, 256)
    P1 = jnp.concatenate([xyz, xyz], axis=-1)
    f1 = _level(X[:, None], Y[:, None], Z[:, None], P1,
                qx1[..., None], qy1[..., None], qz1[..., None], W1, b1)

    qx2, qy2, qz2 = _fps_planes(qx1, qy1, qz1, 128)
    P2 = jnp.concatenate([jnp.stack([qx1, qy1, qz1], axis=-1), f1], axis=-1)
    f2 = _level(qx1[:, None], qy1[:, None], qz1[:, None], P2,
                qx2[..., None], qy2[..., None], qz2[..., None], W2, b2)

    qx3, qy3, qz3 = _fps_planes(qx2, qy2, qz2, 64)
    P3 = jnp.concatenate([jnp.stack([qx2, qy2, qz2], axis=-1), f2], axis=-1)
    f3 = _level(qx2[:, None], qy2[:, None], qz2[:, None], P3,
                qx3[..., None], qy3[..., None], qz3[..., None], W3, b3)

    out = pl.pallas_call(
        _pool_head_kernel,
        out_shape=jax.ShapeDtypeStruct((B, Wf.shape[1]), jnp.float32),
    )(f3, Wf, bf)
    return out


# speedup vs baseline: 1.2322x; 1.2322x over previous
import functools

import jax
import jax.numpy as jnp
from jax import lax
from jax.experimental import pallas as pl
from jax.experimental.pallas import tpu as pltpu
from jax.experimental.pallas import tpu_sc as plsc

K = 32
QC = 4


def _split3(M):
    m1 = M.astype(jnp.bfloat16)
    r1 = M - m1.astype(jnp.float32)
    m2 = r1.astype(jnp.bfloat16)
    m3 = (r1 - m2.astype(jnp.float32)).astype(jnp.bfloat16)
    return m1, m2, m3


def _dotf(a, b):
    return jnp.dot(a, b, preferred_element_type=jnp.float32)


def _fps_body(x_ref, y_ref, z_ref, qx_ref, qy_ref, qz_ref):
    X = x_ref[...]
    Y = y_ref[...]
    Z = z_ref[...]
    B, N = X.shape
    S = qx_ref.shape[1]
    iotaN = jax.lax.broadcasted_iota(jnp.int32, (B, N), 1)
    iotaS = jax.lax.broadcasted_iota(jnp.int32, (B, S), 1)

    def step(i, carry):
        dists, far, qx, qy, qz = carry
        onehot = iotaN == far
        cx = jnp.sum(jnp.where(onehot, X, 0.0), axis=1, keepdims=True)
        cy = jnp.sum(jnp.where(onehot, Y, 0.0), axis=1, keepdims=True)
        cz = jnp.sum(jnp.where(onehot, Z, 0.0), axis=1, keepdims=True)
        d = (X - cx) ** 2 + (Y - cy) ** 2 + (Z - cz) ** 2
        dists = jnp.minimum(dists, d)
        sel = iotaS == i
        qx = jnp.where(sel, cx, qx)
        qy = jnp.where(sel, cy, qy)
        qz = jnp.where(sel, cz, qz)
        m = jnp.max(dists, axis=1, keepdims=True)
        far = jnp.min(jnp.where(dists == m, iotaN, N), axis=1, keepdims=True)
        return dists, far, qx, qy, qz

    dists0 = jnp.full((B, N), 1e10, dtype=jnp.float32)
    far0 = jnp.zeros((B, 1), dtype=jnp.int32)
    q0 = jnp.zeros((B, S), dtype=jnp.float32)
    _, _, qx, qy, qz = jax.lax.fori_loop(0, S, step, (dists0, far0, q0, q0, q0))
    qx_ref[...] = qx
    qy_ref[...] = qy
    qz_ref[...] = qz


def _fps_planes(X, Y, Z, npoints):
    B = X.shape[0]
    shape = jax.ShapeDtypeStruct((B, npoints), jnp.float32)
    return pl.pallas_call(
        _fps_body,
        out_shape=(shape, shape, shape),
    )(X, Y, Z)


def _sel_body(p_ref, x_ref, y_ref, z_ref, qx_ref, qy_ref, qz_ref,
              w_ref, b_ref, a_ref, gidx_ref, cc_ref, d_ref):
    P = p_ref[0]
    X = x_ref[0]
    Y = y_ref[0]
    Z = z_ref[0]
    QX = qx_ref[0]
    QY = qy_ref[0]
    QZ = qz_ref[0]
    W = w_ref[...]
    bvec = b_ref[...]
    S = QX.shape[0]
    N = X.shape[1]

    P1, P2, P3 = _split3(P)
    W1, W2, W3 = _split3(W)
    A = (_dotf(P1, W1)
         + (_dotf(P1, W2) + _dotf(P2, W1))
         + (_dotf(P1, W3) + _dotf(P2, W2) + _dotf(P3, W1)))
    a_ref[0] = A

    d_ref[...] = (QX - X) ** 2 + (QY - Y) ** 2 + (QZ - Z) ** 2
    CH = 128
    NCH = N // CH
    lane_iota = jax.lax.broadcasted_iota(jnp.int32, (S, CH), 1)
    kiota = jax.lax.broadcasted_iota(jnp.int32, (S, K), 1)

    def step(k, carry):
        i_prev, IDX = carry
        mv = jnp.full((S, CH), 1e30, dtype=jnp.float32)
        mi = jnp.full((S, CH), N, dtype=jnp.int32)
        for j in range(NCH):
            ic = lane_iota + (j * CH)
            Dc = jnp.where(ic == i_prev, 1e30,
                           d_ref[:, j * CH:(j + 1) * CH])
            d_ref[:, j * CH:(j + 1) * CH] = Dc
            upd = Dc < mv
            mv = jnp.where(upd, Dc, mv)
            mi = jnp.where(upd, ic, mi)
        m = jnp.min(mv, axis=1, keepdims=True)
        idx = jnp.min(jnp.where(mv == m, mi, N), axis=1, keepdims=True)
        IDX = jnp.where(kiota == k, idx, IDX)
        return idx, IDX

    IDX0 = jnp.zeros((S, K), dtype=jnp.int32)
    i0 = jnp.full((S, 1), -1, dtype=jnp.int32)
    _, IDX = jax.lax.fori_loop(0, K, step, (i0, IDX0))

    b_id = pl.program_id(0)
    gidx_ref[0] = IDX + b_id * N
    C = QX * W[0:1, :] + QY * W[1:2, :] + QZ * W[2:3, :]
    cc_ref[0] = C - bvec


def _pointconv_select(xp, yp, zp, featP, qx, qy, qz, W, b):
    B, _, N = xp.shape
    S = qx.shape[1]
    Cin = featP.shape[2]
    SB = min(S, 128)
    b2d = b.reshape(1, 64)
    return pl.pallas_call(
        _sel_body,
        grid=(B, S // SB),
        in_specs=[
            pl.BlockSpec((1, N, Cin), lambda i, s: (i, 0, 0)),
            pl.BlockSpec((1, 1, N), lambda i, s: (i, 0, 0)),
            pl.BlockSpec((1, 1, N), lambda i, s: (i, 0, 0)),
            pl.BlockSpec((1, 1, N), lambda i, s: (i, 0, 0)),
            pl.BlockSpec((1, SB, 1), lambda i, s: (i, s, 0)),
            pl.BlockSpec((1, SB, 1), lambda i, s: (i, s, 0)),
            pl.BlockSpec((1, SB, 1), lambda i, s: (i, s, 0)),
            pl.BlockSpec((Cin, 64), lambda i, s: (0, 0)),
            pl.BlockSpec((1, 64), lambda i, s: (0, 0)),
        ],
        out_specs=[
            pl.BlockSpec((1, N, 64), lambda i, s: (i, 0, 0)),
            pl.BlockSpec((1, SB, K), lambda i, s: (i, s, 0)),
            pl.BlockSpec((1, SB, 64), lambda i, s: (i, s, 0)),
        ],
        out_shape=[
            jax.ShapeDtypeStruct((B, N, 64), jnp.float32),
            jax.ShapeDtypeStruct((B, S, K), jnp.int32),
            jax.ShapeDtypeStruct((B, S, 64), jnp.float32),
        ],
        scratch_shapes=[pltpu.VMEM((SB, N), jnp.float32)],
    )(featP, xp, yp, zp, qx, qy, qz, W, b2d)


def _make_sc_gather_max(S):
    info = plsc.get_sparse_core_info()
    NC = info.num_cores
    G = S // QC
    mesh = plsc.VectorSubcoreMesh(core_axis_name="c", subcore_axis_name="s")

    @functools.partial(
        pl.kernel, mesh=mesh,
        compiler_params=pltpu.CompilerParams(use_tc_tiling_on_sc=False),
        out_type=jax.ShapeDtypeStruct((32, S, 64), jnp.float32),
        scratch_types=[
            pltpu.VMEM((G, QC * K), jnp.int32),
            pltpu.VMEM((QC * K, 64), jnp.float32),
            pltpu.VMEM((S, 64), jnp.float32),
            pltpu.VMEM((S, 64), jnp.float32),
            pltpu.SemaphoreType.DMA,
        ],
    )
    def k(a_hbm, idx_hbm, cc_hbm, out_hbm, idx_v, rows_v, cc_v, out_v, sem):
        w = lax.axis_index("s") * NC + lax.axis_index("c")
        pltpu.sync_copy(idx_hbm.at[w], idx_v)
        pltpu.sync_copy(cc_hbm.at[w], cc_v)

        def chunk(g, carry):
            pltpu.async_copy(a_hbm.at[idx_v.at[g]], rows_v, sem).wait()
            for q in range(QC):
                for c in range(4):
                    ds = pl.ds(c * 16, 16)
                    acc = rows_v[q * K, ds]
                    for kk in range(1, K):
                        acc = jnp.maximum(acc, rows_v[q * K + kk, ds])
                    qq = g * QC + q
                    out_v[qq, ds] = jnp.maximum(acc - cc_v[qq, ds], 0.0)
            return carry

        lax.fori_loop(0, G, chunk, 0)
        pltpu.sync_copy(out_v, out_hbm.at[w])

    return k


def _pool_head_kernel(f3_ref, wf_ref, bf_ref, out_ref):
    pooled = jnp.mean(f3_ref[...], axis=1)
    out_ref[...] = jnp.dot(pooled, wf_ref[...],
                           precision=jax.lax.Precision.HIGHEST) + bf_ref[0]


def _level(xp, yp, zp, featP, qx, qy, qz, W, b):
    B, _, N = xp.shape
    S = qx.shape[1]
    A, gidx, cc = _pointconv_select(xp, yp, zp, featP, qx, qy, qz, W, b)
    idx3 = gidx.reshape(B, S // QC, QC * K)
    F = _make_sc_gather_max(S)(A.reshape(B * N, 64), idx3, cc)
    return F


def kernel(xyz, W1, b1, W2, b2, W3, b3, Wf, bf, faces):
    B, N, _ = xyz.shape
    X, Y, Z = xyz[..., 0], xyz[..., 1], xyz[..., 2]

    qx1, qy1, qz1 = _fps_planes(X, Y, Z, 256)
    P1 = jnp.concatenate([xyz, xyz], axis=-1)
    f1 = _level(X[:, None], Y[:, None], Z[:, None], P1,
                qx1[..., None], qy1[..., None], qz1[..., None], W1, b1)

    qx2, qy2, qz2 = _fps_planes(qx1, qy1, qz1, 128)
    P2 = jnp.concatenate([jnp.stack([qx1, qy1, qz1], axis=-1), f1], axis=-1)
    f2 = _level(qx1[:, None], qy1[:, None], qz1[:, None], P2,
                qx2[..., None], qy2[..., None], qz2[..., None], W2, b2)

    qx3, qy3, qz3 = _fps_planes(qx2, qy2, qz2, 64)
    P3 = jnp.concatenate([jnp.stack([qx2, qy2, qz2], axis=-1), f2], axis=-1)
    f3 = _level(qx2[:, None], qy2[:, None], qz2[:, None], P3,
                qx3[..., None], qy3[..., None], qz3[..., None], W3, b3)

    out = pl.pallas_call(
        _pool_head_kernel,
        out_shape=jax.ShapeDtypeStruct((B, Wf.shape[1]), jnp.float32),
    )(f3, Wf, bf)
    return out
